# baseline (speedup 1.0000x reference)
import dataclasses
import functools

import jax
import jax.numpy as jnp
from jax import lax
from jax.experimental import pallas as pl
from jax.experimental.pallas import tpu as pltpu
from jax.experimental.pallas import tpu_sc as plsc

N = 10000
N2 = 10016
E = 320000
D = 128
DW = 16
C = 64
NPROC = 168
NWORK = 32
TRIPS = NPROC // 3
SROWS = 12
HALF = SROWS // 2
PAD_ROWS = NWORK * NPROC + SROWS

_SC_COMPILER_PARAMS = pltpu.CompilerParams(use_tc_tiling_on_sc=False)
if "needs_layout_passes" in pltpu.CompilerParams.__dataclass_fields__:
    _SC_COMPILER_PARAMS = dataclasses.replace(_SC_COMPILER_PARAMS, needs_layout_passes=False)

ROWS_PER_TILE = 624
SLAB = 48
DSLAB = 104
TAIL = N - 16 * ROWS_PER_TILE


def _scalar_table_body(fa_ref, fs_ref, fp_ref, lap_ref, rap_ref, lsp_ref, rsp_ref, out_ref):
    fa = fa_ref[...]
    fs = fs_ref[...]
    fp = fp_ref[...]
    el_ap = jnp.sum(fa * lap_ref[...][0][None, :], axis=1)
    er_ap = jnp.sum(fp * rap_ref[...][0][None, :], axis=1)
    el_sp = jnp.sum(fs * lsp_ref[...][0][None, :], axis=1)
    er_sp = jnp.sum(fp * rsp_ref[...][0][None, :], axis=1)
    out_ref[...] = jnp.stack([el_ap, er_ap, el_sp, er_sp], axis=0)


def _scalar_table(feat_author, feat_subject, feat_paper, attn_l_ap, attn_r_ap, attn_l_sp, attn_r_sp):
    return pl.pallas_call(
        _scalar_table_body,
        out_shape=jax.ShapeDtypeStruct((4, N), jnp.float32),
    )(feat_author, feat_subject, feat_paper, attn_l_ap, attn_r_ap, attn_l_sp, attn_r_sp)


def _sc_gat(elr_ap, elr_sp, sd_ap, sd_sp, feat_a, feat_s):
    mesh = plsc.VectorSubcoreMesh(core_axis_name="c", subcore_axis_name="s")

    @functools.partial(
        pl.kernel,
        out_type=[
            jax.ShapeDtypeStruct((2, N, D), jnp.float32),
            jax.ShapeDtypeStruct((2, N, DW), jnp.float32),
            jax.ShapeDtypeStruct((2, N, D), jnp.float32),
            jax.ShapeDtypeStruct((2, N, DW), jnp.float32),
        ],
        mesh=mesh,
        scratch_types=[
            pltpu.VMEM((N2,), jnp.int32),
            pltpu.VMEM((SROWS, C), jnp.int32),
            pltpu.VMEM((6, C), jnp.int32),
            pltpu.VMEM((6, C), jnp.int32),
            pltpu.VMEM((C, D), jnp.float32),
            pltpu.VMEM((C, D), jnp.float32),
            pltpu.VMEM((C, D), jnp.float32),
            pltpu.VMEM((C, DW), jnp.float32),
            pltpu.VMEM((C, DW), jnp.float32),
            pltpu.VMEM((C, DW), jnp.float32),
            pltpu.VMEM((C,), jnp.float32),
            pltpu.VMEM_SHARED((N2, D), jnp.float32),
            pltpu.VMEM_SHARED((N2, DW), jnp.float32),
            pltpu.SemaphoreType.DMA,
            pltpu.SemaphoreType.DMA,
            pltpu.SemaphoreType.DMA,
            pltpu.SemaphoreType.DMA,
            pltpu.SemaphoreType.DMA,
            pltpu.SemaphoreType.DMA,
        ],
        compiler_params=_SC_COMPILER_PARAMS,
    )
    def kern(elr_ap_hbm, elr_sp_hbm, sd_ap_hbm, sd_sp_hbm, feat_a_hbm, feat_s_hbm,
             out_ap_hbm, den_ap_hbm, out_sp_hbm, den_sp_hbm,
             elr_t, stash, sidx, didx, g0, g1, g2, pr0, pr1, pr2, p_col,
             acc, accd, gsA, gsB, gsC, ssem, psem, rsem):
        cid = lax.axis_index("c")
        sid = lax.axis_index("s")
        wid = sid * 2 + cid
        base = wid * NPROC

        gbufs = (g0, g1, g2)
        prbufs = (pr0, pr1, pr2)
        gsems = (gsA, gsB, gsC)

        col_iota = lax.iota(jnp.int32, 16)
        denom_mask = jnp.where(col_iota == 0, 1.0, 0.0).astype(jnp.float32)
        zz = jnp.zeros((16,), jnp.float32)

        def zero_stage0():
            @pl.loop(0, C)
            def _(r):
                for g in range(D // 16):
                    g0[r, pl.ds(16 * g, 16)] = zz
                pr0[r, :] = zz

        def zero_acc():
            for k in range(ROWS_PER_TILE // SLAB):
                r0 = sid * ROWS_PER_TILE + k * SLAB
                pltpu.sync_copy(g0.at[pl.ds(0, SLAB), :], acc.at[pl.ds(r0, SLAB), :])

            for k in range(ROWS_PER_TILE // SLAB):
                r0 = sid * ROWS_PER_TILE + k * SLAB
                pltpu.sync_copy(pr0.at[pl.ds(0, SLAB), :], accd.at[pl.ds(r0, SLAB), :])

            @pl.when(sid == 15)
            def _():
                r0 = 16 * ROWS_PER_TILE
                pltpu.sync_copy(g0.at[pl.ds(0, TAIL), :], acc.at[pl.ds(r0, TAIL), :])
                pltpu.sync_copy(pr0.at[pl.ds(0, TAIL), :], accd.at[pl.ds(r0, TAIL), :])

        def drain(out_hbm, den_hbm):
            for k in range(ROWS_PER_TILE // DSLAB):
                r0 = sid * ROWS_PER_TILE + k * DSLAB
                pltpu.sync_copy(acc.at[pl.ds(r0, DSLAB), :], out_hbm.at[cid].at[pl.ds(r0, DSLAB), :])
                pltpu.sync_copy(accd.at[pl.ds(r0, DSLAB), :], den_hbm.at[cid].at[pl.ds(r0, DSLAB), :])

            @pl.when(sid == 15)
            def _():
                r0 = 16 * ROWS_PER_TILE
                pltpu.sync_copy(acc.at[pl.ds(r0, TAIL), :], out_hbm.at[cid].at[pl.ds(r0, TAIL), :])
                pltpu.sync_copy(accd.at[pl.ds(r0, TAIL), :], den_hbm.at[cid].at[pl.ds(r0, TAIL), :])

        def run_relation(elr_hbm, sd_hbm, feat_hbm, out_hbm, den_hbm):
            pltpu.sync_copy(elr_hbm, elr_t)

            def store_idx(row, slot):
                for g in range(C // 16):
                    sd = stash[row, pl.ds(16 * g, 16)]
                    sidx[slot, pl.ds(16 * g, 16)] = sd & 0xFFFF
                    didx[slot, pl.ds(16 * g, 16)] = sd >> 16

            def gather_start(slot, gref, gsem):
                pltpu.async_copy(feat_hbm.at[sidx.at[slot]], gref, gsem)

            def gather_wait(slot, gref, gsem):
                pltpu.make_async_copy(feat_hbm.at[sidx.at[slot]], gref, gsem).wait()

            def scatter_start(slot, gref, pref):
                pltpu.async_copy(gref, acc.at[didx.at[slot]], ssem, add=True)
                pltpu.async_copy(pref, accd.at[didx.at[slot]], psem, add=True)

            def scatter_wait(slot, gref, pref):
                pltpu.make_async_copy(gref, acc.at[didx.at[slot]], ssem).wait()
                pltpu.make_async_copy(pref, accd.at[didx.at[slot]], psem).wait()

            def refill_start(win):
                h0 = lax.rem(win, 2) * HALF
                pltpu.async_copy(sd_hbm.at[pl.ds(base + win * HALF, HALF), :],
                                 stash.at[pl.ds(h0, HALF), :], rsem)

            def refill_wait(win):
                h0 = lax.rem(win, 2) * HALF
                pltpu.make_async_copy(sd_hbm.at[pl.ds(base + win * HALF, HALF), :],
                                      stash.at[pl.ds(h0, HALF), :], rsem).wait()

            def compute(row, gref, pref):
                for g in range(C // 16):
                    sd = stash[row, pl.ds(16 * g, 16)]
                    ts = plsc.load_gather(elr_t, [sd & 0xFFFF])
                    td = plsc.load_gather(elr_t, [sd >> 16])
                    el = plsc.bitcast(ts << 16, jnp.float32)
                    er = plsc.bitcast(td & jnp.int32(-65536), jnp.float32)
                    e = el + er
                    e = jnp.where(e >= 0.0, e, 0.01 * e)
                    p_col[pl.ds(16 * g, 16)] = jnp.exp(e)

                @pl.loop(0, C, unroll=4)
                def _(r):
                    psp = plsc.load_gather(p_col, [jnp.full((16,), r, jnp.int32)])
                    for g in range(D // 16):
                        gref[r, pl.ds(16 * g, 16)] = gref[r, pl.ds(16 * g, 16)] * psp
                    pref[r, :] = psp * denom_mask

            pltpu.sync_copy(sd_hbm.at[pl.ds(base, 2 * HALF), :], stash)
            store_idx(0, 0)
            gather_start(0, g0, gsA)
            store_idx(1, 1)
            gather_start(1, g1, gsB)

            @pl.loop(0, TRIPS)
            def _(u):
                um = lax.rem(u, 2)
                slot_base = 3 * um
                for k in range(3):
                    j = 3 * u + k
                    row = lax.rem(j, SROWS)
                    slot = slot_base + k
                    gref = gbufs[k]
                    pref = prbufs[k]

                    gather_wait(slot, gref, gsems[k])
                    compute(row, gref, pref)

                    if k == 0:

                        @pl.when(u > 0)
                        def _():
                            scatter_wait(5 - slot_base, gbufs[2], prbufs[2])
                    else:
                        scatter_wait(slot - 1, gbufs[k - 1], prbufs[k - 1])

                    scatter_start(slot, gref, pref)

                    if k == 1:
                        @pl.when((um == 1) & (u > 1) & (u < TRIPS - 1))
                        def _():
                            refill_wait(lax.div(u + 1, 2))

                    @pl.when(j + 2 < NPROC)
                    def _():
                        store_idx(lax.rem(j + 2, SROWS), lax.rem(j + 2, 6))
                        gather_start(lax.rem(j + 2, 6), gbufs[(k + 2) % 3],
                                     gsems[(k + 2) % 3])

                    if k == 2:

                        @pl.when((um == 1) & (u < TRIPS - 3))
                        def _():
                            refill_start(lax.div(u + 1, 2) + 1)

            scatter_wait((NPROC - 1) % 6, gbufs[2], prbufs[2])

        zero_stage0()
        zero_acc()
        plsc.subcore_barrier()
        run_relation(elr_ap_hbm, sd_ap_hbm, feat_a_hbm, out_ap_hbm, den_ap_hbm)
        plsc.subcore_barrier()
        drain(out_ap_hbm, den_ap_hbm)
        zero_stage0()
        zero_acc()
        plsc.subcore_barrier()
        run_relation(elr_sp_hbm, sd_sp_hbm, feat_s_hbm, out_sp_hbm, den_sp_hbm)
        plsc.subcore_barrier()
        drain(out_sp_hbm, den_sp_hbm)

    return kern(elr_ap, elr_sp, sd_ap, sd_sp, feat_a, feat_s)


ROWS_TC = 1000
GRID_TC = N // ROWS_TC


def _post_body(ap_ref, dap_ref, sp_ref, dsp_ref, fcw_ref, fcb_ref,
               h_ap_ref, h_sp_ref, tsum_ref):
    step = pl.program_id(0)

    @pl.when(step == 0)
    def _():
        tsum_ref[...] = jnp.zeros_like(tsum_ref)

    fcw = fcw_ref[...]
    fcb = fcb_ref[...]
    for m, (part_ref, den_ref, h_ref) in enumerate(
            ((ap_ref, dap_ref, h_ap_ref), (sp_ref, dsp_ref, h_sp_ref))):
        num = part_ref[...][0] + part_ref[...][1]
        denf = den_ref[...][0] + den_ref[...][1]
        den = denf[:, 0:1]
        h = jnp.where(den > 0.0, num / jnp.where(den > 0.0, den, 1.0), 0.0)
        h = jnp.where(h > 0.0, h, jnp.exp(h) - 1.0)
        h_ref[...] = h
        t = jnp.tanh(
            jax.lax.dot_general(h, fcw, (((1,), (1,)), ((), ())),
                                preferred_element_type=jnp.float32) + fcb[None, :])
        tsum_ref[pl.ds(m, 1), :] += jnp.sum(t, axis=0, keepdims=True)


def _post(out_ap, den_ap, out_sp, den_sp, fc_w, fc_b):
    return pl.pallas_call(
        _post_body,
        grid=(GRID_TC,),
        in_specs=[
            pl.BlockSpec((2, ROWS_TC, D), lambda i: (0, i, 0)),
            pl.BlockSpec((2, ROWS_TC, DW), lambda i: (0, i, 0)),
            pl.BlockSpec((2, ROWS_TC, D), lambda i: (0, i, 0)),
            pl.BlockSpec((2, ROWS_TC, DW), lambda i: (0, i, 0)),
            pl.BlockSpec((D, D), lambda i: (0, 0)),
            pl.BlockSpec((D,), lambda i: (0,)),
        ],
        out_specs=[
            pl.BlockSpec((ROWS_TC, D), lambda i: (i, 0)),
            pl.BlockSpec((ROWS_TC, D), lambda i: (i, 0)),
            pl.BlockSpec((2, D), lambda i: (0, 0)),
        ],
        out_shape=[
            jax.ShapeDtypeStruct((N, D), jnp.float32),
            jax.ShapeDtypeStruct((N, D), jnp.float32),
            jax.ShapeDtypeStruct((2, D), jnp.float32),
        ],
    )(out_ap, den_ap, out_sp, den_sp, fc_w, fc_b)


def _combine_body(h_ap_ref, h_sp_ref, tsum_ref, sem_ref, out_ref):
    tmean = tsum_ref[...] * (1.0 / N)
    a = sem_ref[...][0]
    w0 = jnp.sum(tmean[0] * a)
    w1 = jnp.sum(tmean[1] * a)
    m = jnp.maximum(w0, w1)
    b0 = jnp.exp(w0 - m)
    b1 = jnp.exp(w1 - m)
    s = b0 + b1
    out_ref[...] = (b0 * h_ap_ref[...] + b1 * h_sp_ref[...]) / s


def _combine(h_ap, h_sp, tsum, attn_sem):
    return pl.pallas_call(
        _combine_body,
        grid=(GRID_TC,),
        in_specs=[
            pl.BlockSpec((ROWS_TC, D), lambda i: (i, 0)),
            pl.BlockSpec((ROWS_TC, D), lambda i: (i, 0)),
            pl.BlockSpec((2, D), lambda i: (0, 0)),
            pl.BlockSpec((1, D), lambda i: (0, 0)),
        ],
        out_specs=pl.BlockSpec((ROWS_TC, D), lambda i: (i, 0)),
        out_shape=jax.ShapeDtypeStruct((N, D), jnp.float32),
    )(h_ap, h_sp, tsum, attn_sem)


def _pack_edges(edge_index):
    sd = edge_index[0] + (edge_index[1] << 16)
    sd = sd.reshape(E // C, C)
    pad_val = jnp.int32(N + (N << 16))
    return jnp.pad(sd, ((0, PAD_ROWS - E // C), (0, 0)), constant_values=pad_val)


def _pack_elr(el, er):
    def bits(x):
        return jax.lax.bitcast_convert_type(x.astype(jnp.bfloat16), jnp.uint16).astype(jnp.int32)

    elr = bits(el) | (bits(er) << 16)
    return jnp.pad(elr, (0, N2 - N))


def kernel(feat_author, feat_subject, feat_paper, edge_index_ap, edge_index_sp,
           attn_l_ap, attn_r_ap, attn_l_sp, attn_r_sp, fc_w, fc_b, attn_sem):
    scal = _scalar_table(feat_author, feat_subject, feat_paper,
                         attn_l_ap, attn_r_ap, attn_l_sp, attn_r_sp)
    elr_ap = _pack_elr(scal[0], scal[1])
    elr_sp = _pack_elr(scal[2], scal[3])
    sd_ap = _pack_edges(edge_index_ap)
    sd_sp = _pack_edges(edge_index_sp)
    feat_a = jnp.pad(feat_author, ((0, N2 - N), (0, 0)))
    feat_s = jnp.pad(feat_subject, ((0, N2 - N), (0, 0)))
    out_ap, den_ap, out_sp, den_sp = _sc_gat(elr_ap, elr_sp, sd_ap, sd_sp, feat_a, feat_s)
    h_ap, h_sp, tsum = _post(out_ap, den_ap, out_sp, den_sp, fc_w, fc_b)
    return _combine(h_ap, h_sp, tsum, attn_sem)


# speedup vs baseline: 1.8225x; 1.8225x over previous
import dataclasses
import functools

import jax
import jax.numpy as jnp
from jax import lax
from jax.experimental import pallas as pl
from jax.experimental.pallas import tpu as pltpu
from jax.experimental.pallas import tpu_sc as plsc

N = 10000
E = 320000
D = 128
DW = 16
C = 128
HS = 64
NCHUNK = E // C
NWORK = 32
BASE_CHUNKS = NCHUNK // NWORK
EXTRA = NCHUNK - BASE_CHUNKS * NWORK
B = 4
NBATCH = (BASE_CHUNKS + 1 + B - 1) // B
PAD_CHUNKS = NCHUNK + B

_SC_COMPILER_PARAMS = pltpu.CompilerParams(use_tc_tiling_on_sc=False)
if "needs_layout_passes" in pltpu.CompilerParams.__dataclass_fields__:
    _SC_COMPILER_PARAMS = dataclasses.replace(_SC_COMPILER_PARAMS, needs_layout_passes=False)

ROWS_PER_TILE = 624
SLAB = 104
TAIL_ROWS = N - 16 * ROWS_PER_TILE


def _scalar_table_body(fa_ref, fs_ref, fp_ref, lap_ref, rap_ref, lsp_ref, rsp_ref, out_ref):
    fa = fa_ref[...]
    fs = fs_ref[...]
    fp = fp_ref[...]
    el_ap = jnp.sum(fa * lap_ref[...][0][None, :], axis=1)
    er_ap = jnp.sum(fp * rap_ref[...][0][None, :], axis=1)
    el_sp = jnp.sum(fs * lsp_ref[...][0][None, :], axis=1)
    er_sp = jnp.sum(fp * rsp_ref[...][0][None, :], axis=1)
    out_ref[...] = jnp.stack([el_ap, er_ap, el_sp, er_sp], axis=0)


def _scalar_table(feat_author, feat_subject, feat_paper, attn_l_ap, attn_r_ap, attn_l_sp, attn_r_sp):
    return pl.pallas_call(
        _scalar_table_body,
        out_shape=jax.ShapeDtypeStruct((4, N), jnp.float32),
    )(feat_author, feat_subject, feat_paper, attn_l_ap, attn_r_ap, attn_l_sp, attn_r_sp)


def _sc_gat(scal, src_ap, dst_ap, src_sp, dst_sp, feat_a, feat_s):
    mesh = plsc.VectorSubcoreMesh(core_axis_name="c", subcore_axis_name="s")

    @functools.partial(
        pl.kernel,
        out_type=[
            jax.ShapeDtypeStruct((2, N, D), jnp.float32),
            jax.ShapeDtypeStruct((2, N, DW), jnp.float32),
            jax.ShapeDtypeStruct((2, N, D), jnp.float32),
            jax.ShapeDtypeStruct((2, N, DW), jnp.float32),
        ],
        mesh=mesh,
        scratch_types=[
            pltpu.VMEM((N,), jnp.float32),
            pltpu.VMEM((N,), jnp.float32),
            pltpu.VMEM((B, C), jnp.int32),
            pltpu.VMEM((B, C), jnp.int32),
            pltpu.VMEM((C, D), jnp.float32),
            pltpu.VMEM((C, DW), jnp.float32),
            pltpu.VMEM((C,), jnp.float32),
            pltpu.VMEM_SHARED((N, D), jnp.float32),
            pltpu.VMEM_SHARED((N, DW), jnp.float32),
            pltpu.SemaphoreType.DMA,
            pltpu.SemaphoreType.DMA,
            pltpu.SemaphoreType.DMA,
        ],
        compiler_params=_SC_COMPILER_PARAMS,
    )
    def kern(scal_hbm, src_ap_hbm, dst_ap_hbm, src_sp_hbm, dst_sp_hbm,
             feat_a_hbm, feat_s_hbm, out_ap_hbm, den_ap_hbm, out_sp_hbm, den_sp_hbm,
             el_t, er_t, src_t, dst_t, rows_g, p_rows, p_col, acc, accd,
             gs0, gs1, psem):
        cid = lax.axis_index("c")
        sid = lax.axis_index("s")
        wid = sid * 2 + cid
        n_my = BASE_CHUNKS + jnp.where(wid < EXTRA, 1, 0)
        cstart = wid * BASE_CHUNKS + jnp.minimum(wid, EXTRA)

        col_iota = lax.iota(jnp.int32, 16)
        denom_mask = jnp.where(col_iota == 0, 1.0, 0.0).astype(jnp.float32)
        zz = jnp.zeros((16,), jnp.float32)

        def zero_local():
            @pl.loop(0, C)
            def _(r):
                for g in range(D // 16):
                    rows_g[r, pl.ds(16 * g, 16)] = zz
                p_rows[r, :] = zz

        def zero_acc():
            for k in range(ROWS_PER_TILE // SLAB):
                r0 = sid * ROWS_PER_TILE + k * SLAB
                pltpu.sync_copy(rows_g.at[pl.ds(0, SLAB), :], acc.at[pl.ds(r0, SLAB), :])
                pltpu.sync_copy(p_rows.at[pl.ds(0, SLAB), :], accd.at[pl.ds(r0, SLAB), :])

            @pl.when(sid == 15)
            def _():
                r0 = 16 * ROWS_PER_TILE
                pltpu.sync_copy(rows_g.at[pl.ds(0, TAIL_ROWS), :], acc.at[pl.ds(r0, TAIL_ROWS), :])
                pltpu.sync_copy(p_rows.at[pl.ds(0, TAIL_ROWS), :], accd.at[pl.ds(r0, TAIL_ROWS), :])

        def drain(out_hbm, den_hbm):
            for k in range(ROWS_PER_TILE // SLAB):
                r0 = sid * ROWS_PER_TILE + k * SLAB
                pltpu.sync_copy(acc.at[pl.ds(r0, SLAB), :], out_hbm.at[cid].at[pl.ds(r0, SLAB), :])
                pltpu.sync_copy(accd.at[pl.ds(r0, SLAB), :], den_hbm.at[cid].at[pl.ds(r0, SLAB), :])

            @pl.when(sid == 15)
            def _():
                r0 = 16 * ROWS_PER_TILE
                pltpu.sync_copy(acc.at[pl.ds(r0, TAIL_ROWS), :], out_hbm.at[cid].at[pl.ds(r0, TAIL_ROWS), :])
                pltpu.sync_copy(accd.at[pl.ds(r0, TAIL_ROWS), :], den_hbm.at[cid].at[pl.ds(r0, TAIL_ROWS), :])

        def run_relation(src_hbm, dst_hbm, feat_hbm, el_row, er_row):
            pltpu.sync_copy(scal_hbm.at[el_row], el_t)
            pltpu.sync_copy(scal_hbm.at[er_row], er_t)

            def p_scatter_wait(j):
                pltpu.make_async_copy(p_rows, accd.at[dst_t.at[j]], psem).wait()

            @pl.loop(0, NBATCH)
            def _(b):
                b0 = cstart + b * B
                @pl.when(b > 0)
                def _():
                    p_scatter_wait(B - 1)

                pltpu.sync_copy(src_hbm.at[pl.ds(b0, B), :], src_t)
                pltpu.sync_copy(dst_hbm.at[pl.ds(b0, B), :], dst_t)
                jcount = jnp.minimum(n_my - b * B, B)

                @pl.loop(0, jcount)
                def _(j):
                    pltpu.async_copy(feat_hbm.at[src_t.at[j, pl.ds(0, HS)]],
                                     rows_g.at[pl.ds(0, HS), :], gs0)
                    pltpu.async_copy(feat_hbm.at[src_t.at[j, pl.ds(HS, HS)]],
                                     rows_g.at[pl.ds(HS, HS), :], gs1)

                    for g in range(C // 16):
                        sv = src_t[j, pl.ds(16 * g, 16)]
                        dv = dst_t[j, pl.ds(16 * g, 16)]
                        e = plsc.load_gather(el_t, [sv]) + plsc.load_gather(er_t, [dv])
                        e = jnp.where(e >= 0.0, e, 0.01 * e)
                        p_col[pl.ds(16 * g, 16)] = jnp.exp(e)

                    @pl.when(j > 0)
                    def _():
                        p_scatter_wait(j - 1)

                    def scale(lo):
                        @pl.loop(lo, lo + HS, unroll=4)
                        def _(r):
                            psp = plsc.load_gather(p_col, [jnp.full((16,), r, jnp.int32)])
                            for g in range(D // 16):
                                rows_g[r, pl.ds(16 * g, 16)] = rows_g[r, pl.ds(16 * g, 16)] * psp
                            p_rows[r, :] = psp * denom_mask

                    pltpu.make_async_copy(feat_hbm.at[src_t.at[j, pl.ds(0, HS)]],
                                          rows_g.at[pl.ds(0, HS), :], gs0).wait()
                    scale(0)
                    pltpu.make_async_copy(feat_hbm.at[src_t.at[j, pl.ds(HS, HS)]],
                                          rows_g.at[pl.ds(HS, HS), :], gs1).wait()
                    scale(HS)

                    pltpu.sync_copy(rows_g, acc.at[dst_t.at[j]], add=True)
                    pltpu.async_copy(p_rows, accd.at[dst_t.at[j]], psem, add=True)

            p_scatter_wait(jnp.minimum(n_my - (NBATCH - 1) * B, B) - 1)

        zero_local()
        zero_acc()
        plsc.subcore_barrier()
        run_relation(src_ap_hbm, dst_ap_hbm, feat_a_hbm, 0, 1)
        plsc.subcore_barrier()
        drain(out_ap_hbm, den_ap_hbm)
        zero_local()
        zero_acc()
        plsc.subcore_barrier()
        run_relation(src_sp_hbm, dst_sp_hbm, feat_s_hbm, 2, 3)
        plsc.subcore_barrier()
        drain(out_sp_hbm, den_sp_hbm)

    return kern(scal, src_ap, dst_ap, src_sp, dst_sp, feat_a, feat_s)


ROWS_TC = 1000
GRID_TC = N // ROWS_TC


def _post_body(ap_ref, dap_ref, sp_ref, dsp_ref, fcw_ref, fcb_ref,
               h_ap_ref, h_sp_ref, tsum_ref):
    step = pl.program_id(0)

    @pl.when(step == 0)
    def _():
        tsum_ref[...] = jnp.zeros_like(tsum_ref)

    fcw = fcw_ref[...]
    fcb = fcb_ref[...]
    for m, (part_ref, den_ref, h_ref) in enumerate(
            ((ap_ref, dap_ref, h_ap_ref), (sp_ref, dsp_ref, h_sp_ref))):
        num = part_ref[...][0] + part_ref[...][1]
        denf = den_ref[...][0] + den_ref[...][1]
        den = denf[:, 0:1]
        h = jnp.where(den > 0.0, num / jnp.where(den > 0.0, den, 1.0), 0.0)
        h = jnp.where(h > 0.0, h, jnp.exp(h) - 1.0)
        h_ref[...] = h
        t = jnp.tanh(
            jax.lax.dot_general(h, fcw, (((1,), (1,)), ((), ())),
                                preferred_element_type=jnp.float32) + fcb[None, :])
        tsum_ref[pl.ds(m, 1), :] += jnp.sum(t, axis=0, keepdims=True)


def _post(out_ap, den_ap, out_sp, den_sp, fc_w, fc_b):
    return pl.pallas_call(
        _post_body,
        grid=(GRID_TC,),
        in_specs=[
            pl.BlockSpec((2, ROWS_TC, D), lambda i: (0, i, 0)),
            pl.BlockSpec((2, ROWS_TC, DW), lambda i: (0, i, 0)),
            pl.BlockSpec((2, ROWS_TC, D), lambda i: (0, i, 0)),
            pl.BlockSpec((2, ROWS_TC, DW), lambda i: (0, i, 0)),
            pl.BlockSpec((D, D), lambda i: (0, 0)),
            pl.BlockSpec((D,), lambda i: (0,)),
        ],
        out_specs=[
            pl.BlockSpec((ROWS_TC, D), lambda i: (i, 0)),
            pl.BlockSpec((ROWS_TC, D), lambda i: (i, 0)),
            pl.BlockSpec((2, D), lambda i: (0, 0)),
        ],
        out_shape=[
            jax.ShapeDtypeStruct((N, D), jnp.float32),
            jax.ShapeDtypeStruct((N, D), jnp.float32),
            jax.ShapeDtypeStruct((2, D), jnp.float32),
        ],
    )(out_ap, den_ap, out_sp, den_sp, fc_w, fc_b)


def _combine_body(h_ap_ref, h_sp_ref, tsum_ref, sem_ref, out_ref):
    tmean = tsum_ref[...] * (1.0 / N)
    a = sem_ref[...][0]
    w0 = jnp.sum(tmean[0] * a)
    w1 = jnp.sum(tmean[1] * a)
    m = jnp.maximum(w0, w1)
    b0 = jnp.exp(w0 - m)
    b1 = jnp.exp(w1 - m)
    s = b0 + b1
    out_ref[...] = (b0 * h_ap_ref[...] + b1 * h_sp_ref[...]) / s


def _combine(h_ap, h_sp, tsum, attn_sem):
    return pl.pallas_call(
        _combine_body,
        grid=(GRID_TC,),
        in_specs=[
            pl.BlockSpec((ROWS_TC, D), lambda i: (i, 0)),
            pl.BlockSpec((ROWS_TC, D), lambda i: (i, 0)),
            pl.BlockSpec((2, D), lambda i: (0, 0)),
            pl.BlockSpec((1, D), lambda i: (0, 0)),
        ],
        out_specs=pl.BlockSpec((ROWS_TC, D), lambda i: (i, 0)),
        out_shape=jax.ShapeDtypeStruct((N, D), jnp.float32),
    )(h_ap, h_sp, tsum, attn_sem)


def _pad_chunks(x):
    return jnp.pad(x.reshape(NCHUNK, C), ((0, PAD_CHUNKS - NCHUNK), (0, 0)))


def kernel(feat_author, feat_subject, feat_paper, edge_index_ap, edge_index_sp,
           attn_l_ap, attn_r_ap, attn_l_sp, attn_r_sp, fc_w, fc_b, attn_sem):
    scal = _scalar_table(feat_author, feat_subject, feat_paper,
                         attn_l_ap, attn_r_ap, attn_l_sp, attn_r_sp)
    src_ap = _pad_chunks(edge_index_ap[0])
    dst_ap = _pad_chunks(edge_index_ap[1])
    src_sp = _pad_chunks(edge_index_sp[0])
    dst_sp = _pad_chunks(edge_index_sp[1])
    out_ap, den_ap, out_sp, den_sp = _sc_gat(scal, src_ap, dst_ap, src_sp, dst_sp,
                                             feat_author, feat_subject)
    h_ap, h_sp, tsum = _post(out_ap, den_ap, out_sp, den_sp, fc_w, fc_b)
    return _combine(h_ap, h_sp, tsum, attn_sem)


# speedup vs baseline: 2.8185x; 1.5465x over previous
import dataclasses
import functools

import jax
import jax.numpy as jnp
from jax import lax
from jax.experimental import pallas as pl
from jax.experimental.pallas import tpu as pltpu
from jax.experimental.pallas import tpu_sc as plsc

N = 10000
E = 320000
D = 128
DW = 16
C = 128
HS = 64
NCHUNK = E // C
NWORK = 32
BASE_CHUNKS = NCHUNK // NWORK
EXTRA = NCHUNK - BASE_CHUNKS * NWORK
B = 4
NBATCH = (BASE_CHUNKS + 1 + B - 1) // B
PAD_CHUNKS = NCHUNK + B

_SC_COMPILER_PARAMS = pltpu.CompilerParams(use_tc_tiling_on_sc=False)
if "needs_layout_passes" in pltpu.CompilerParams.__dataclass_fields__:
    _SC_COMPILER_PARAMS = dataclasses.replace(_SC_COMPILER_PARAMS, needs_layout_passes=False)

ROWS_PER_TILE = 624
SLAB = 104
TAIL_ROWS = N - 16 * ROWS_PER_TILE


def _scalar_table_body(fa_ref, fs_ref, fp_ref, lap_ref, rap_ref, lsp_ref, rsp_ref, out_ref):
    fa = fa_ref[...]
    fs = fs_ref[...]
    fp = fp_ref[...]
    el_ap = jnp.sum(fa * lap_ref[...][0][None, :], axis=1)
    er_ap = jnp.sum(fp * rap_ref[...][0][None, :], axis=1)
    el_sp = jnp.sum(fs * lsp_ref[...][0][None, :], axis=1)
    er_sp = jnp.sum(fp * rsp_ref[...][0][None, :], axis=1)
    out_ref[...] = jnp.stack([el_ap, er_ap, el_sp, er_sp], axis=0)


def _scalar_table(feat_author, feat_subject, feat_paper, attn_l_ap, attn_r_ap, attn_l_sp, attn_r_sp):
    return pl.pallas_call(
        _scalar_table_body,
        out_shape=jax.ShapeDtypeStruct((4, N), jnp.float32),
    )(feat_author, feat_subject, feat_paper, attn_l_ap, attn_r_ap, attn_l_sp, attn_r_sp)


def _sc_gat(scal, src_ap, dst_ap, src_sp, dst_sp, feat_a, feat_s):
    mesh = plsc.VectorSubcoreMesh(core_axis_name="c", subcore_axis_name="s")

    @functools.partial(
        pl.kernel,
        out_type=[
            jax.ShapeDtypeStruct((2, N, D), jnp.float32),
            jax.ShapeDtypeStruct((2, N, DW), jnp.float32),
            jax.ShapeDtypeStruct((2, N, D), jnp.float32),
            jax.ShapeDtypeStruct((2, N, DW), jnp.float32),
        ],
        mesh=mesh,
        scratch_types=[
            pltpu.VMEM((N,), jnp.float32),
            pltpu.VMEM((N,), jnp.float32),
            pltpu.VMEM((B, C), jnp.int32),
            pltpu.VMEM((B, C), jnp.int32),
            pltpu.VMEM((C, D), jnp.float32),
            pltpu.VMEM((C, DW), jnp.float32),
            pltpu.VMEM((C,), jnp.float32),
            pltpu.VMEM_SHARED((N, D), jnp.float32),
            pltpu.VMEM_SHARED((N, DW), jnp.float32),
            pltpu.SemaphoreType.DMA,
            pltpu.SemaphoreType.DMA,
            pltpu.SemaphoreType.DMA,
        ],
        compiler_params=_SC_COMPILER_PARAMS,
    )
    def kern(scal_hbm, src_ap_hbm, dst_ap_hbm, src_sp_hbm, dst_sp_hbm,
             feat_a_hbm, feat_s_hbm, out_ap_hbm, den_ap_hbm, out_sp_hbm, den_sp_hbm,
             el_t, er_t, src_t, dst_t, rows_g, p_rows, p_col, acc, accd,
             gs0, gs1, psem):
        cid = lax.axis_index("c")
        sid = lax.axis_index("s")
        wid = sid * 2 + cid
        n_my = BASE_CHUNKS + jnp.where(wid < EXTRA, 1, 0)
        cstart = wid * BASE_CHUNKS + jnp.minimum(wid, EXTRA)

        col_iota = lax.iota(jnp.int32, 16)
        denom_mask = jnp.where(col_iota == 0, 1.0, 0.0).astype(jnp.float32)
        zz = jnp.zeros((16,), jnp.float32)

        def zero_local():
            @pl.loop(0, C)
            def _(r):
                for g in range(D // 16):
                    rows_g[r, pl.ds(16 * g, 16)] = zz
                p_rows[r, :] = zz

        def zero_acc():
            for k in range(ROWS_PER_TILE // SLAB):
                r0 = sid * ROWS_PER_TILE + k * SLAB
                pltpu.sync_copy(rows_g.at[pl.ds(0, SLAB), :], acc.at[pl.ds(r0, SLAB), :])
                pltpu.sync_copy(p_rows.at[pl.ds(0, SLAB), :], accd.at[pl.ds(r0, SLAB), :])

            @pl.when(sid == 15)
            def _():
                r0 = 16 * ROWS_PER_TILE
                pltpu.sync_copy(rows_g.at[pl.ds(0, TAIL_ROWS), :], acc.at[pl.ds(r0, TAIL_ROWS), :])
                pltpu.sync_copy(p_rows.at[pl.ds(0, TAIL_ROWS), :], accd.at[pl.ds(r0, TAIL_ROWS), :])

        def drain(out_hbm, den_hbm):
            for k in range(ROWS_PER_TILE // SLAB):
                r0 = sid * ROWS_PER_TILE + k * SLAB
                pltpu.sync_copy(acc.at[pl.ds(r0, SLAB), :], out_hbm.at[cid].at[pl.ds(r0, SLAB), :])
                pltpu.sync_copy(accd.at[pl.ds(r0, SLAB), :], den_hbm.at[cid].at[pl.ds(r0, SLAB), :])

            @pl.when(sid == 15)
            def _():
                r0 = 16 * ROWS_PER_TILE
                pltpu.sync_copy(acc.at[pl.ds(r0, TAIL_ROWS), :], out_hbm.at[cid].at[pl.ds(r0, TAIL_ROWS), :])
                pltpu.sync_copy(accd.at[pl.ds(r0, TAIL_ROWS), :], den_hbm.at[cid].at[pl.ds(r0, TAIL_ROWS), :])

        def run_relation(src_hbm, dst_hbm, feat_hbm, el_row, er_row):
            pltpu.sync_copy(scal_hbm.at[el_row], el_t)
            pltpu.sync_copy(scal_hbm.at[er_row], er_t)

            @pl.loop(0, NBATCH)
            def _(b):
                b0 = cstart + b * B
                pltpu.sync_copy(src_hbm.at[pl.ds(b0, B), :], src_t)
                pltpu.sync_copy(dst_hbm.at[pl.ds(b0, B), :], dst_t)
                jcount = jnp.minimum(n_my - b * B, B)

                @pl.loop(0, jcount)
                def _(j):
                    pltpu.async_copy(feat_hbm.at[src_t.at[j]], rows_g, gs0).wait()
                    for g in range(C // 16):
                        sv = src_t[j, pl.ds(16 * g, 16)]
                        dv = dst_t[j, pl.ds(16 * g, 16)]
                        e = plsc.load_gather(el_t, [sv]) + plsc.load_gather(er_t, [dv])
                        e = jnp.where(e >= 0.0, e, 0.01 * e)
                        p_col[pl.ds(16 * g, 16)] = jnp.exp(e)

                    @pl.loop(0, C)
                    def _(r):
                        psp = plsc.load_gather(p_col, [jnp.full((16,), r, jnp.int32)])
                        for g in range(D // 16):
                            rows_g[r, pl.ds(16 * g, 16)] = rows_g[r, pl.ds(16 * g, 16)] * psp
                        p_rows[r, :] = psp * denom_mask

                    pltpu.sync_copy(rows_g, acc.at[dst_t.at[j]], add=True)
                    pltpu.sync_copy(p_rows, accd.at[dst_t.at[j]], add=True)

        zero_local()
        zero_acc()
        plsc.subcore_barrier()
        run_relation(src_ap_hbm, dst_ap_hbm, feat_a_hbm, 0, 1)
        plsc.subcore_barrier()
        drain(out_ap_hbm, den_ap_hbm)
        zero_local()
        zero_acc()
        plsc.subcore_barrier()
        run_relation(src_sp_hbm, dst_sp_hbm, feat_s_hbm, 2, 3)
        plsc.subcore_barrier()
        drain(out_sp_hbm, den_sp_hbm)

    return kern(scal, src_ap, dst_ap, src_sp, dst_sp, feat_a, feat_s)


ROWS_TC = 1000
GRID_TC = N // ROWS_TC


def _post_body(ap_ref, dap_ref, sp_ref, dsp_ref, fcw_ref, fcb_ref,
               h_ap_ref, h_sp_ref, tsum_ref):
    step = pl.program_id(0)

    @pl.when(step == 0)
    def _():
        tsum_ref[...] = jnp.zeros_like(tsum_ref)

    fcw = fcw_ref[...]
    fcb = fcb_ref[...]
    for m, (part_ref, den_ref, h_ref) in enumerate(
            ((ap_ref, dap_ref, h_ap_ref), (sp_ref, dsp_ref, h_sp_ref))):
        num = part_ref[...][0] + part_ref[...][1]
        denf = den_ref[...][0] + den_ref[...][1]
        den = denf[:, 0:1]
        h = jnp.where(den > 0.0, num / jnp.where(den > 0.0, den, 1.0), 0.0)
        h = jnp.where(h > 0.0, h, jnp.exp(h) - 1.0)
        h_ref[...] = h
        t = jnp.tanh(
            jax.lax.dot_general(h, fcw, (((1,), (1,)), ((), ())),
                                preferred_element_type=jnp.float32) + fcb[None, :])
        tsum_ref[pl.ds(m, 1), :] += jnp.sum(t, axis=0, keepdims=True)


def _post(out_ap, den_ap, out_sp, den_sp, fc_w, fc_b):
    return pl.pallas_call(
        _post_body,
        grid=(GRID_TC,),
        in_specs=[
            pl.BlockSpec((2, ROWS_TC, D), lambda i: (0, i, 0)),
            pl.BlockSpec((2, ROWS_TC, DW), lambda i: (0, i, 0)),
            pl.BlockSpec((2, ROWS_TC, D), lambda i: (0, i, 0)),
            pl.BlockSpec((2, ROWS_TC, DW), lambda i: (0, i, 0)),
            pl.BlockSpec((D, D), lambda i: (0, 0)),
            pl.BlockSpec((D,), lambda i: (0,)),
        ],
        out_specs=[
            pl.BlockSpec((ROWS_TC, D), lambda i: (i, 0)),
            pl.BlockSpec((ROWS_TC, D), lambda i: (i, 0)),
            pl.BlockSpec((2, D), lambda i: (0, 0)),
        ],
        out_shape=[
            jax.ShapeDtypeStruct((N, D), jnp.float32),
            jax.ShapeDtypeStruct((N, D), jnp.float32),
            jax.ShapeDtypeStruct((2, D), jnp.float32),
        ],
    )(out_ap, den_ap, out_sp, den_sp, fc_w, fc_b)


def _combine_body(h_ap_ref, h_sp_ref, tsum_ref, sem_ref, out_ref):
    tmean = tsum_ref[...] * (1.0 / N)
    a = sem_ref[...][0]
    w0 = jnp.sum(tmean[0] * a)
    w1 = jnp.sum(tmean[1] * a)
    m = jnp.maximum(w0, w1)
    b0 = jnp.exp(w0 - m)
    b1 = jnp.exp(w1 - m)
    s = b0 + b1
    out_ref[...] = (b0 * h_ap_ref[...] + b1 * h_sp_ref[...]) / s


def _combine(h_ap, h_sp, tsum, attn_sem):
    return pl.pallas_call(
        _combine_body,
        grid=(GRID_TC,),
        in_specs=[
            pl.BlockSpec((ROWS_TC, D), lambda i: (i, 0)),
            pl.BlockSpec((ROWS_TC, D), lambda i: (i, 0)),
            pl.BlockSpec((2, D), lambda i: (0, 0)),
            pl.BlockSpec((1, D), lambda i: (0, 0)),
        ],
        out_specs=pl.BlockSpec((ROWS_TC, D), lambda i: (i, 0)),
        out_shape=jax.ShapeDtypeStruct((N, D), jnp.float32),
    )(h_ap, h_sp, tsum, attn_sem)


def _pad_chunks(x):
    return jnp.pad(x.reshape(NCHUNK, C), ((0, PAD_CHUNKS - NCHUNK), (0, 0)))


def kernel(feat_author, feat_subject, feat_paper, edge_index_ap, edge_index_sp,
           attn_l_ap, attn_r_ap, attn_l_sp, attn_r_sp, fc_w, fc_b, attn_sem):
    scal = _scalar_table(feat_author, feat_subject, feat_paper,
                         attn_l_ap, attn_r_ap, attn_l_sp, attn_r_sp)
    src_ap = _pad_chunks(edge_index_ap[0])
    dst_ap = _pad_chunks(edge_index_ap[1])
    src_sp = _pad_chunks(edge_index_sp[0])
    dst_sp = _pad_chunks(edge_index_sp[1])
    out_ap, den_ap, out_sp, den_sp = _sc_gat(scal, src_ap, dst_ap, src_sp, dst_sp,
                                             feat_author, feat_subject)
    h_ap, h_sp, tsum = _post(out_ap, den_ap, out_sp, den_sp, fc_w, fc_b)
    return _combine(h_ap, h_sp, tsum, attn_sem)
